# baseline (speedup 1.0000x reference)
import functools

import jax
import jax.numpy as jnp
from jax.experimental import pallas as pl

B = 256
NN = 5
H = 50
CATN = 18
C = CATN + 1
D = 400
AD = 128
NODES = H + CATN
L = 2
R = NN * C
BB = 4

_INV_SCALE = 1.0 / (AD ** 0.5)


def _dotx(x, w, dn):
    return jax.lax.dot_general(x, w, dn, preferred_element_type=jnp.float32,
                               precision=jax.lax.Precision.HIGHEST)


def _dot(x, w):
    return jax.lax.dot_general(x, w, (((x.ndim - 1,), (0,)), ((), ())),
                               preferred_element_type=jnp.float32)


def _sue_kernel(h0_ref, cand_ref, graph_ref, maskf_ref, idx_ref,
                Wg_ref, bg_ref, WK_ref, WQ_ref, bQ_ref,
                Waff_ref, baff_ref, Wck_ref, Wcq_ref, bcq_ref, out_ref):
    h0 = h0_ref[...]
    g = h0
    for l in range(L):
        agg = jnp.stack([jnp.dot(graph_ref[u], g[u],
                                 preferred_element_type=jnp.float32)
                         for u in range(BB)])
        g = g + jax.nn.relu(_dot(agg, Wg_ref[l]) + bg_ref[l])
    gf = (g + h0)[:, :H, :]

    K3 = _dot(gf, WK_ref[...])
    cand = cand_ref[...]
    Q3 = _dot(cand, WQ_ref[...]) + bQ_ref[...]
    Qc3 = _dot(cand, Wcq_ref[...]) + bcq_ref[...]

    row_iota = jax.lax.broadcasted_iota(jnp.int32, (R, NN), 0)
    col_iota = jax.lax.broadcasted_iota(jnp.int32, (R, NN), 1)
    Pn = (row_iota // C == col_iota).astype(jnp.float32)
    rowc_iota = jax.lax.broadcasted_iota(jnp.int32, (R, C), 0)
    cc_iota = jax.lax.broadcasted_iota(jnp.int32, (R, C), 1)
    Pc = (rowc_iota % C == cc_iota).astype(jnp.float32)
    nmask = (jax.lax.broadcasted_iota(jnp.int32, (NN, R), 0)
             == jax.lax.broadcasted_iota(jnp.int32, (NN, R), 1) // C
             ).astype(jnp.float32)

    intra_list = []
    for u in range(BB):
        a = jax.lax.dot_general(
            Q3[u], K3[u], (((1,), (1,)), ((), ())),
            preferred_element_type=jnp.float32) * _INV_SCALE

        idx_u = idx_ref[u]
        cat_iota = jax.lax.broadcasted_iota(jnp.int32, (C, H), 0)
        onehot = (cat_iota == idx_u).astype(jnp.float32)

        masked = jnp.where(onehot[None, :, :] > 0, a[:, None, :], -1e30)
        M = jnp.max(masked, axis=-1)
        m_h = _dotx(M, onehot, (((1,), (0,)), ((), ())))
        ex = jnp.exp(a - m_h)
        ssum = _dotx(ex, onehot, (((1,), (1,)), ((), ())))
        denom = _dotx(ssum, onehot, (((1,), (0,)), ((), ()))) + 1e-12
        alpha = ex / denom

        cfull = ((jax.lax.broadcasted_iota(jnp.int32, (R, H), 0) % C)
                 == idx_u).astype(jnp.float32)
        alphaR = _dotx(Pn, alpha, (((1,), (0,)), ((), ())))
        intra_list.append(_dotx(cfull * alphaR, gf[u],
                                (((1,), (0,)), ((), ()))))

    intra = jnp.stack(intra_list)
    intra2 = jax.nn.relu(_dot(intra, Waff_ref[...]) + baff_ref[...]) + intra
    Kc3 = _dot(intra2, Wck_ref[...])

    for u in range(BB):
        E = jax.lax.dot_general(
            Qc3[u], Kc3[u], (((1,), (1,)), ((), ())),
            preferred_element_type=jnp.float32)
        e = _dotx(E * nmask, Pc, (((1,), (0,)), ((), ()))) * _INV_SCALE
        e = jnp.where(maskf_ref[u] > 0, e, -1e9)
        e = e - jnp.max(e, axis=-1, keepdims=True)
        we = jnp.exp(e)
        w = we / jnp.sum(we, axis=-1, keepdims=True)
        wR = _dotx(w, Pc, (((1,), (1,)), ((), ()))) * nmask
        out_ref[u] = _dotx(wR, intra2[u], (((1,), (0,)), ((), ())))


@jax.jit
def _sue_pallas(h0, cand, graph, maskf, idx, W_gcn, b_gcn, W_K,
                W_Q, b_Q, W_aff, b_aff, W_ck, W_cq, b_cq):
    grid = (B // BB,)
    data_spec3 = lambda s1, s2: pl.BlockSpec((BB, s1, s2), lambda i: (i, 0, 0))
    w_spec = lambda shape: pl.BlockSpec(shape, lambda i: (0,) * len(shape))
    return pl.pallas_call(
        _sue_kernel,
        grid=grid,
        in_specs=[
            data_spec3(NODES, D),
            data_spec3(NN, D),
            data_spec3(NODES, NODES),
            data_spec3(1, C),
            data_spec3(1, H),
            w_spec((L, D, D)),
            w_spec((L, 1, D)),
            w_spec((D, AD)),
            w_spec((D, AD)),
            w_spec((1, AD)),
            w_spec((D, D)),
            w_spec((1, D)),
            w_spec((D, AD)),
            w_spec((D, AD)),
            w_spec((1, AD)),
        ],
        out_specs=data_spec3(NN, D),
        out_shape=jax.ShapeDtypeStruct((B, NN, D), jnp.float32),
    )(h0, cand, graph, maskf, idx, W_gcn, b_gcn, W_K, W_Q, b_Q,
      W_aff, b_aff, W_ck, W_cq, b_cq)


def kernel(history_embedding, candidate_news_representation, user_history_graph,
           user_history_category_mask, user_history_category_indices,
           proxy_node_embedding, W_gcn, b_gcn, W_K, W_Q, b_Q, W_aff, b_aff,
           W_ck, W_cq, b_cq):
    h0 = jnp.concatenate(
        [history_embedding,
         jnp.broadcast_to(proxy_node_embedding[None], (B, CATN, D))], axis=1)
    maskf = user_history_category_mask.at[:, -1].set(1)
    maskf = (maskf > 0).astype(jnp.float32).reshape(B, 1, C)
    idx = user_history_category_indices.astype(jnp.int32).reshape(B, 1, H)
    return _sue_pallas(
        h0, candidate_news_representation, user_history_graph,
        maskf, idx, W_gcn, b_gcn.reshape(L, 1, D), W_K, W_Q,
        b_Q.reshape(1, AD), W_aff, b_aff.reshape(1, D), W_ck, W_cq,
        b_cq.reshape(1, AD))


# speedup vs baseline: 1.0434x; 1.0434x over previous
import jax
import jax.numpy as jnp
from jax.experimental import pallas as pl

B = 256
NN = 5
H = 50
CATN = 18
C = CATN + 1
D = 400
AD = 128
NODES = H + CATN
NP = 72
L = 2
R = 96
BB = 4

_INV_SCALE = 1.0 / (AD ** 0.5)


def _dotx(x, w, dn):
    return jax.lax.dot_general(x, w, dn, preferred_element_type=jnp.float32,
                               precision=jax.lax.Precision.HIGHEST)


def _dot2(x, w):
    return jax.lax.dot_general(x, w, (((1,), (0,)), ((), ())),
                               preferred_element_type=jnp.float32)


def _sue_kernel(h0_ref, cand_ref, graph_ref, maskf_ref, onehot_ref,
                Pn_ref, Pc_ref, nmask_ref,
                Wg_ref, bg_ref, WK_ref, WQ_ref, bQ_ref,
                Waff_ref, baff_ref, Wck_ref, Wcq_ref, bcq_ref, out_ref):
    h0 = h0_ref[...].reshape(BB * NP, D)
    g = h0
    for l in range(L):
        agg = jnp.concatenate(
            [_dot2(graph_ref[u], g[u * NP:(u + 1) * NP]) for u in range(BB)],
            axis=0)
        g = g + jax.nn.relu(_dot2(agg, Wg_ref[l]) + bg_ref[l])
    gfa = g + h0

    K_all = _dot2(gfa, WK_ref[...])
    cand = cand_ref[...].reshape(BB * NN, D)
    Q_all = _dot2(cand, WQ_ref[...]) + bQ_ref[...]
    Qc_all = _dot2(cand, Wcq_ref[...]) + bcq_ref[...]

    Pn = Pn_ref[...]
    Pc = Pc_ref[...]
    nmask = nmask_ref[...]

    intra_list = []
    for u in range(BB):
        K_u = K_all[u * NP:u * NP + H]
        Q_u = Q_all[u * NN:(u + 1) * NN]
        a = jax.lax.dot_general(
            Q_u, K_u, (((1,), (1,)), ((), ())),
            preferred_element_type=jnp.float32) * _INV_SCALE

        onehot = onehot_ref[u]
        masked = jnp.where(onehot[None, :, :] > 0, a[:, None, :], -1e30)
        M = jnp.max(masked, axis=-1)
        m_h = _dotx(M, onehot, (((1,), (0,)), ((), ())))
        ex = jnp.exp(a - m_h)
        ssum = _dotx(ex, onehot, (((1,), (1,)), ((), ())))
        denom = _dotx(ssum, onehot, (((1,), (0,)), ((), ()))) + 1e-12
        alpha = ex / denom

        cfull = jnp.dot(Pc, onehot,
                        preferred_element_type=jnp.float32)
        alphaR = _dotx(Pn, alpha, (((1,), (0,)), ((), ())))
        gf_u = gfa[u * NP:u * NP + H]
        intra_list.append(_dotx(cfull * alphaR, gf_u,
                                (((1,), (0,)), ((), ()))))

    intra = jnp.concatenate(intra_list, axis=0)
    intra2 = jax.nn.relu(_dot2(intra, Waff_ref[...]) + baff_ref[...]) + intra
    Kc_all = _dot2(intra2, Wck_ref[...])

    for u in range(BB):
        Kc_u = Kc_all[u * R:(u + 1) * R]
        Qc_u = Qc_all[u * NN:(u + 1) * NN]
        E = jax.lax.dot_general(
            Qc_u, Kc_u, (((1,), (1,)), ((), ())),
            preferred_element_type=jnp.float32)
        e = _dotx(E * nmask, Pc, (((1,), (0,)), ((), ()))) * _INV_SCALE
        e = jnp.where(maskf_ref[u] > 0, e, -1e9)
        e = e - jnp.max(e, axis=-1, keepdims=True)
        we = jnp.exp(e)
        w = we / jnp.sum(we, axis=-1, keepdims=True)
        wR = _dotx(w, Pc, (((1,), (1,)), ((), ()))) * nmask
        out_ref[u] = _dotx(wR, intra2[u * R:(u + 1) * R],
                           (((1,), (0,)), ((), ())))


@jax.jit
def _sue_pallas(h0, cand, graph, maskf, onehot, Pn, Pc, nmask,
                W_gcn, b_gcn, W_K, W_Q, b_Q, W_aff, b_aff, W_ck, W_cq, b_cq):
    grid = (B // BB,)
    data_spec3 = lambda s1, s2: pl.BlockSpec((BB, s1, s2), lambda i: (i, 0, 0))
    w_spec = lambda shape: pl.BlockSpec(shape, lambda i: (0,) * len(shape))
    return pl.pallas_call(
        _sue_kernel,
        grid=grid,
        in_specs=[
            data_spec3(NP, D),
            data_spec3(NN, D),
            data_spec3(NP, NP),
            data_spec3(1, C),
            data_spec3(C, H),
            w_spec((R, NN)),
            w_spec((R, C)),
            w_spec((NN, R)),
            w_spec((L, D, D)),
            w_spec((L, 1, D)),
            w_spec((D, AD)),
            w_spec((D, AD)),
            w_spec((1, AD)),
            w_spec((D, D)),
            w_spec((1, D)),
            w_spec((D, AD)),
            w_spec((D, AD)),
            w_spec((1, AD)),
        ],
        out_specs=data_spec3(NN, D),
        out_shape=jax.ShapeDtypeStruct((B, NN, D), jnp.float32),
    )(h0, cand, graph, maskf, onehot, Pn, Pc, nmask,
      W_gcn, b_gcn, W_K, W_Q, b_Q, W_aff, b_aff, W_ck, W_cq, b_cq)


def kernel(history_embedding, candidate_news_representation, user_history_graph,
           user_history_category_mask, user_history_category_indices,
           proxy_node_embedding, W_gcn, b_gcn, W_K, W_Q, b_Q, W_aff, b_aff,
           W_ck, W_cq, b_cq):
    h0 = jnp.concatenate(
        [history_embedding,
         jnp.broadcast_to(proxy_node_embedding[None], (B, CATN, D)),
         jnp.zeros((B, NP - NODES, D), jnp.float32)], axis=1)
    graph = jnp.pad(user_history_graph,
                    ((0, 0), (0, NP - NODES), (0, NP - NODES)))
    maskf = user_history_category_mask.at[:, -1].set(1)
    maskf = (maskf > 0).astype(jnp.float32).reshape(B, 1, C)
    idx = user_history_category_indices.astype(jnp.int32)
    onehot = (idx[:, None, :] == jnp.arange(C, dtype=jnp.int32)[None, :, None]
              ).astype(jnp.float32)
    r = jnp.arange(R, dtype=jnp.int32)
    valid = (r < NN * C).astype(jnp.float32)
    Pn = ((r[:, None] // C == jnp.arange(NN, dtype=jnp.int32)[None, :])
          .astype(jnp.float32) * valid[:, None])
    Pc = ((r[:, None] % C == jnp.arange(C, dtype=jnp.int32)[None, :])
          .astype(jnp.float32) * valid[:, None])
    nmask = Pn.T
    return _sue_pallas(
        h0, candidate_news_representation, graph,
        maskf, onehot, Pn, Pc, nmask,
        W_gcn, b_gcn.reshape(L, 1, D), W_K, W_Q,
        b_Q.reshape(1, AD), W_aff, b_aff.reshape(1, D), W_ck, W_cq,
        b_cq.reshape(1, AD))


# speedup vs baseline: 1.1123x; 1.0660x over previous
import jax
import jax.numpy as jnp
from jax.experimental import pallas as pl

B = 256
NN = 5
H = 50
CATN = 18
C = CATN + 1
D = 400
AD = 128
NODES = H + CATN
NP = 72
L = 2
R = 96
BB = 8

_INV_SCALE = 1.0 / (AD ** 0.5)


def _dotx(x, w, dn):
    return jax.lax.dot_general(x, w, dn, preferred_element_type=jnp.float32,
                               precision=jax.lax.Precision.HIGHEST)


def _dot2(x, w):
    return jax.lax.dot_general(x, w, (((1,), (0,)), ((), ())),
                               preferred_element_type=jnp.float32)


def _sue_kernel(h0_ref, cand_ref, graph_ref, maskf_ref, onehot_ref,
                Pn_ref, Pc_ref, nmask_ref,
                Wg_ref, bg_ref, WK_ref, WQ_ref, bQ_ref,
                Waff_ref, baff_ref, Wck_ref, Wcq_ref, bcq_ref, out_ref):
    h0 = h0_ref[...].reshape(BB * NP, D)
    g = h0
    for l in range(L):
        agg = jnp.concatenate(
            [_dot2(graph_ref[u], g[u * NP:(u + 1) * NP]) for u in range(BB)],
            axis=0)
        g = g + jax.nn.relu(_dot2(agg, Wg_ref[l]) + bg_ref[l])
    gfa = g + h0

    K_all = _dot2(gfa, WK_ref[...])
    cand = cand_ref[...].reshape(BB * NN, D)
    Q_all = _dot2(cand, WQ_ref[...]) + bQ_ref[...]
    Qc_all = _dot2(cand, Wcq_ref[...]) + bcq_ref[...]

    Pn = Pn_ref[...]
    Pc = Pc_ref[...]
    nmask = nmask_ref[...]

    intra_list = []
    for u in range(BB):
        K_u = K_all[u * NP:u * NP + H]
        Q_u = Q_all[u * NN:(u + 1) * NN]
        a = jax.lax.dot_general(
            Q_u, K_u, (((1,), (1,)), ((), ())),
            preferred_element_type=jnp.float32) * _INV_SCALE

        onehot = onehot_ref[u]
        masked = jnp.where(onehot[None, :, :] > 0, a[:, None, :], -1e30)
        M = jnp.max(masked, axis=-1)
        m_h = _dotx(M, onehot, (((1,), (0,)), ((), ())))
        ex = jnp.exp(a - m_h)
        ssum = _dotx(ex, onehot, (((1,), (1,)), ((), ())))
        denom = _dotx(ssum, onehot, (((1,), (0,)), ((), ()))) + 1e-12
        alpha = ex / denom

        cfull = jnp.dot(Pc, onehot,
                        preferred_element_type=jnp.float32)
        alphaR = _dotx(Pn, alpha, (((1,), (0,)), ((), ())))
        gf_u = gfa[u * NP:u * NP + H]
        intra_list.append(_dotx(cfull * alphaR, gf_u,
                                (((1,), (0,)), ((), ()))))

    intra = jnp.concatenate(intra_list, axis=0)
    intra2 = jax.nn.relu(_dot2(intra, Waff_ref[...]) + baff_ref[...]) + intra
    Kc_all = _dot2(intra2, Wck_ref[...])

    for u in range(BB):
        Kc_u = Kc_all[u * R:(u + 1) * R]
        Qc_u = Qc_all[u * NN:(u + 1) * NN]
        E = jax.lax.dot_general(
            Qc_u, Kc_u, (((1,), (1,)), ((), ())),
            preferred_element_type=jnp.float32)
        e = _dotx(E * nmask, Pc, (((1,), (0,)), ((), ()))) * _INV_SCALE
        e = jnp.where(maskf_ref[u] > 0, e, -1e9)
        e = e - jnp.max(e, axis=-1, keepdims=True)
        we = jnp.exp(e)
        w = we / jnp.sum(we, axis=-1, keepdims=True)
        wR = _dotx(w, Pc, (((1,), (1,)), ((), ()))) * nmask
        out_ref[u] = _dotx(wR, intra2[u * R:(u + 1) * R],
                           (((1,), (0,)), ((), ())))


@jax.jit
def _sue_pallas(h0, cand, graph, maskf, onehot, Pn, Pc, nmask,
                W_gcn, b_gcn, W_K, W_Q, b_Q, W_aff, b_aff, W_ck, W_cq, b_cq):
    grid = (B // BB,)
    data_spec3 = lambda s1, s2: pl.BlockSpec((BB, s1, s2), lambda i: (i, 0, 0))
    w_spec = lambda shape: pl.BlockSpec(shape, lambda i: (0,) * len(shape))
    return pl.pallas_call(
        _sue_kernel,
        grid=grid,
        in_specs=[
            data_spec3(NP, D),
            data_spec3(NN, D),
            data_spec3(NP, NP),
            data_spec3(1, C),
            data_spec3(C, H),
            w_spec((R, NN)),
            w_spec((R, C)),
            w_spec((NN, R)),
            w_spec((L, D, D)),
            w_spec((L, 1, D)),
            w_spec((D, AD)),
            w_spec((D, AD)),
            w_spec((1, AD)),
            w_spec((D, D)),
            w_spec((1, D)),
            w_spec((D, AD)),
            w_spec((D, AD)),
            w_spec((1, AD)),
        ],
        out_specs=data_spec3(NN, D),
        out_shape=jax.ShapeDtypeStruct((B, NN, D), jnp.float32),
    )(h0, cand, graph, maskf, onehot, Pn, Pc, nmask,
      W_gcn, b_gcn, W_K, W_Q, b_Q, W_aff, b_aff, W_ck, W_cq, b_cq)


def kernel(history_embedding, candidate_news_representation, user_history_graph,
           user_history_category_mask, user_history_category_indices,
           proxy_node_embedding, W_gcn, b_gcn, W_K, W_Q, b_Q, W_aff, b_aff,
           W_ck, W_cq, b_cq):
    h0 = jnp.concatenate(
        [history_embedding,
         jnp.broadcast_to(proxy_node_embedding[None], (B, CATN, D)),
         jnp.zeros((B, NP - NODES, D), jnp.float32)], axis=1)
    graph = jnp.pad(user_history_graph,
                    ((0, 0), (0, NP - NODES), (0, NP - NODES)))
    maskf = user_history_category_mask.at[:, -1].set(1)
    maskf = (maskf > 0).astype(jnp.float32).reshape(B, 1, C)
    idx = user_history_category_indices.astype(jnp.int32)
    onehot = (idx[:, None, :] == jnp.arange(C, dtype=jnp.int32)[None, :, None]
              ).astype(jnp.float32)
    r = jnp.arange(R, dtype=jnp.int32)
    valid = (r < NN * C).astype(jnp.float32)
    Pn = ((r[:, None] // C == jnp.arange(NN, dtype=jnp.int32)[None, :])
          .astype(jnp.float32) * valid[:, None])
    Pc = ((r[:, None] % C == jnp.arange(C, dtype=jnp.int32)[None, :])
          .astype(jnp.float32) * valid[:, None])
    nmask = Pn.T
    return _sue_pallas(
        h0, candidate_news_representation, graph,
        maskf, onehot, Pn, Pc, nmask,
        W_gcn, b_gcn.reshape(L, 1, D), W_K, W_Q,
        b_Q.reshape(1, AD), W_aff, b_aff.reshape(1, D), W_ck, W_cq,
        b_cq.reshape(1, AD))


# speedup vs baseline: 1.2533x; 1.1268x over previous
import jax
import jax.numpy as jnp
from jax.experimental import pallas as pl

B = 256
NN = 5
H = 50
CATN = 18
C = CATN + 1
D = 400
AD = 128
NODES = H + CATN
NP = 72
L = 2
R = 96
BB = 8

_INV_SCALE = 1.0 / (AD ** 0.5)


def _dg(x, w, dn):
    return jax.lax.dot_general(x, w, dn, preferred_element_type=jnp.float32)


def _split(x):
    hi = x.astype(jnp.bfloat16).astype(jnp.float32)
    return hi, x - hi


def _dot_splitL(x, oh, dn):
    x_hi, x_lo = _split(x)
    return _dg(x_hi, oh, dn) + _dg(x_lo, oh, dn)


def _dot2(x, w):
    return jax.lax.dot_general(x, w, (((1,), (0,)), ((), ())),
                               preferred_element_type=jnp.float32)


def _sue_kernel(h0_ref, cand_ref, graph_ref, maskf_ref, onehot_ref,
                Pn_ref, Pc_ref, nmask_ref,
                Wg_ref, bg_ref, WK_ref, WQ_ref, bQ_ref,
                Waff_ref, baff_ref, Wck_ref, Wcq_ref, bcq_ref, out_ref):
    h0 = h0_ref[...].reshape(BB * NP, D)
    g = h0
    for l in range(L):
        agg = jnp.concatenate(
            [_dot2(graph_ref[u], g[u * NP:(u + 1) * NP]) for u in range(BB)],
            axis=0)
        g = g + jax.nn.relu(_dot2(agg, Wg_ref[l]) + bg_ref[l])
    gfa = g + h0
    gfa_hi, gfa_lo = _split(gfa)

    K_all = _dot2(gfa, WK_ref[...])
    cand = cand_ref[...].reshape(BB * NN, D)
    Q_all = _dot2(cand, WQ_ref[...]) + bQ_ref[...]
    Qc_all = _dot2(cand, Wcq_ref[...]) + bcq_ref[...]

    Pn = Pn_ref[...]
    Pc = Pc_ref[...]
    nmask = nmask_ref[...]

    intra_list = []
    for u in range(BB):
        K_u = K_all[u * NP:u * NP + H]
        Q_u = Q_all[u * NN:(u + 1) * NN]
        a = jax.lax.dot_general(
            Q_u, K_u, (((1,), (1,)), ((), ())),
            preferred_element_type=jnp.float32) * _INV_SCALE

        onehot = onehot_ref[u]
        masked = jnp.where(onehot[None, :, :] > 0, a[:, None, :], -1e30)
        M = jnp.max(masked, axis=-1)
        m_h = _dot_splitL(M, onehot, (((1,), (0,)), ((), ())))
        ex = jnp.exp(a - m_h)
        ssum = _dot_splitL(ex, onehot, (((1,), (1,)), ((), ())))
        denom = _dot_splitL(ssum, onehot, (((1,), (0,)), ((), ()))) + 1e-12
        alpha = ex / denom

        cfull = jnp.dot(Pc, onehot,
                        preferred_element_type=jnp.float32)
        a_hi, a_lo = _split(alpha)
        dnL = (((1,), (0,)), ((), ()))
        wf_hi = _dg(Pn, a_hi, dnL) * cfull
        wf_lo = _dg(Pn, a_lo, dnL) * cfull
        gf_hi = gfa_hi[u * NP:u * NP + H]
        gf_lo = gfa_lo[u * NP:u * NP + H]
        intra_list.append(_dg(wf_hi, gf_hi, dnL) + _dg(wf_hi, gf_lo, dnL)
                          + _dg(wf_lo, gf_hi, dnL))

    intra = jnp.concatenate(intra_list, axis=0)
    intra2 = jax.nn.relu(_dot2(intra, Waff_ref[...]) + baff_ref[...]) + intra
    Kc_all = _dot2(intra2, Wck_ref[...])
    i2_hi, i2_lo = _split(intra2)

    for u in range(BB):
        Kc_u = Kc_all[u * R:(u + 1) * R]
        Qc_u = Qc_all[u * NN:(u + 1) * NN]
        E = jax.lax.dot_general(
            Qc_u, Kc_u, (((1,), (1,)), ((), ())),
            preferred_element_type=jnp.float32)
        dnL = (((1,), (0,)), ((), ()))
        e = _dot_splitL(E * nmask, Pc, dnL) * _INV_SCALE
        e = jnp.where(maskf_ref[u] > 0, e, -1e9)
        e = e - jnp.max(e, axis=-1, keepdims=True)
        we = jnp.exp(e)
        w = we / jnp.sum(we, axis=-1, keepdims=True)
        w_hi, w_lo = _split(w)
        dnT = (((1,), (1,)), ((), ()))
        wR_hi = _dg(w_hi, Pc, dnT) * nmask
        wR_lo = _dg(w_lo, Pc, dnT) * nmask
        i2u_hi = i2_hi[u * R:(u + 1) * R]
        i2u_lo = i2_lo[u * R:(u + 1) * R]
        out_ref[u] = (_dg(wR_hi, i2u_hi, dnL) + _dg(wR_hi, i2u_lo, dnL)
                      + _dg(wR_lo, i2u_hi, dnL))


@jax.jit
def _sue_pallas(h0, cand, graph, maskf, onehot, Pn, Pc, nmask,
                W_gcn, b_gcn, W_K, W_Q, b_Q, W_aff, b_aff, W_ck, W_cq, b_cq):
    grid = (B // BB,)
    data_spec3 = lambda s1, s2: pl.BlockSpec((BB, s1, s2), lambda i: (i, 0, 0))
    w_spec = lambda shape: pl.BlockSpec(shape, lambda i: (0,) * len(shape))
    return pl.pallas_call(
        _sue_kernel,
        grid=grid,
        in_specs=[
            data_spec3(NP, D),
            data_spec3(NN, D),
            data_spec3(NP, NP),
            data_spec3(1, C),
            data_spec3(C, H),
            w_spec((R, NN)),
            w_spec((R, C)),
            w_spec((NN, R)),
            w_spec((L, D, D)),
            w_spec((L, 1, D)),
            w_spec((D, AD)),
            w_spec((D, AD)),
            w_spec((1, AD)),
            w_spec((D, D)),
            w_spec((1, D)),
            w_spec((D, AD)),
            w_spec((D, AD)),
            w_spec((1, AD)),
        ],
        out_specs=data_spec3(NN, D),
        out_shape=jax.ShapeDtypeStruct((B, NN, D), jnp.float32),
    )(h0, cand, graph, maskf, onehot, Pn, Pc, nmask,
      W_gcn, b_gcn, W_K, W_Q, b_Q, W_aff, b_aff, W_ck, W_cq, b_cq)


def kernel(history_embedding, candidate_news_representation, user_history_graph,
           user_history_category_mask, user_history_category_indices,
           proxy_node_embedding, W_gcn, b_gcn, W_K, W_Q, b_Q, W_aff, b_aff,
           W_ck, W_cq, b_cq):
    h0 = jnp.concatenate(
        [history_embedding,
         jnp.broadcast_to(proxy_node_embedding[None], (B, CATN, D)),
         jnp.zeros((B, NP - NODES, D), jnp.float32)], axis=1)
    graph = jnp.pad(user_history_graph,
                    ((0, 0), (0, NP - NODES), (0, NP - NODES)))
    maskf = user_history_category_mask.at[:, -1].set(1)
    maskf = (maskf > 0).astype(jnp.float32).reshape(B, 1, C)
    idx = user_history_category_indices.astype(jnp.int32)
    onehot = (idx[:, None, :] == jnp.arange(C, dtype=jnp.int32)[None, :, None]
              ).astype(jnp.float32)
    r = jnp.arange(R, dtype=jnp.int32)
    valid = (r < NN * C).astype(jnp.float32)
    Pn = ((r[:, None] // C == jnp.arange(NN, dtype=jnp.int32)[None, :])
          .astype(jnp.float32) * valid[:, None])
    Pc = ((r[:, None] % C == jnp.arange(C, dtype=jnp.int32)[None, :])
          .astype(jnp.float32) * valid[:, None])
    nmask = Pn.T
    return _sue_pallas(
        h0, candidate_news_representation, graph,
        maskf, onehot, Pn, Pc, nmask,
        W_gcn, b_gcn.reshape(L, 1, D), W_K, W_Q,
        b_Q.reshape(1, AD), W_aff, b_aff.reshape(1, D), W_ck, W_cq,
        b_cq.reshape(1, AD))


# speedup vs baseline: 2.7577x; 2.2003x over previous
import jax
import jax.numpy as jnp
from jax.experimental import pallas as pl

B = 256
NN = 5
H = 50
CATN = 18
C = CATN + 1
D = 400
AD = 128
NODES = H + CATN
NP = 72
L = 2
R = 96
BB = 8

_INV_SCALE = 1.0 / (AD ** 0.5)


def _dg(x, w, dn):
    return jax.lax.dot_general(x, w, dn, preferred_element_type=jnp.float32)


def _split(x):
    hi = x.astype(jnp.bfloat16).astype(jnp.float32)
    return hi, x - hi


def _dot_splitL(x, oh, dn):
    x_hi, x_lo = _split(x)
    return _dg(x_hi, oh, dn) + _dg(x_lo, oh, dn)


def _dot2(x, w):
    return jax.lax.dot_general(x, w, (((1,), (0,)), ((), ())),
                               preferred_element_type=jnp.float32)


def _sue_kernel(h0_ref, cand_ref, graph_ref, maskf_ref, onehot_ref,
                Pn_ref, Pc_ref, nmask_ref,
                Wg_ref, bg_ref, WK_ref, WQ_ref, bQ_ref,
                Waff_ref, baff_ref, Wck_ref, Wcq_ref, bcq_ref, out_ref):
    h0 = h0_ref[...].reshape(BB * NP, D)
    g = h0
    for l in range(L):
        agg = jnp.concatenate(
            [_dot2(graph_ref[u], g[u * NP:(u + 1) * NP]) for u in range(BB)],
            axis=0)
        g = g + jax.nn.relu(_dot2(agg, Wg_ref[l]) + bg_ref[l])
    gfa = g + h0
    gfa_hi, gfa_lo = _split(gfa)

    K_all = _dot2(gfa, WK_ref[...])
    cand = cand_ref[...].reshape(BB * NN, D)
    Q_all = _dot2(cand, WQ_ref[...]) + bQ_ref[...]
    Qc_all = _dot2(cand, Wcq_ref[...]) + bcq_ref[...]

    Pn = Pn_ref[...]
    Pc = Pc_ref[...]
    nmask = nmask_ref[...]
    dnL = (((1,), (0,)), ((), ()))
    dnT = (((1,), (1,)), ((), ()))
    US = range(BB)

    a_l = [_dg(Q_all[u * NN:(u + 1) * NN],
               K_all[u * NP:u * NP + H], dnT) * _INV_SCALE for u in US]
    oh_l = [onehot_ref[u] for u in US]
    M_l = [jnp.max(jnp.where(oh_l[u][None, :, :] > 0,
                             a_l[u][:, None, :], -1e30), axis=-1) for u in US]
    m_l = [_dg(M_l[u], oh_l[u], dnL) for u in US]
    ex_l = [jnp.exp(a_l[u] - m_l[u]) for u in US]
    exs_l = [_split(ex_l[u]) for u in US]
    ssum_l = [_dg(exs_l[u][0], oh_l[u], dnT)
              + _dg(exs_l[u][1], oh_l[u], dnT) for u in US]
    sss_l = [_split(ssum_l[u]) for u in US]
    den_l = [_dg(sss_l[u][0], oh_l[u], dnL)
             + _dg(sss_l[u][1], oh_l[u], dnL) + 1e-12 for u in US]
    al_l = [_split(ex_l[u] / den_l[u]) for u in US]
    cf_l = [_dg(Pc, oh_l[u], dnL) for u in US]
    wfh_l = [_dg(Pn, al_l[u][0], dnL) * cf_l[u] for u in US]
    wfl_l = [_dg(Pn, al_l[u][1], dnL) * cf_l[u] for u in US]
    intra_list = [
        _dg(wfh_l[u], gfa_hi[u * NP:u * NP + H], dnL)
        + _dg(wfh_l[u], gfa_lo[u * NP:u * NP + H], dnL)
        + _dg(wfl_l[u], gfa_hi[u * NP:u * NP + H], dnL) for u in US]

    intra = jnp.concatenate(intra_list, axis=0)
    intra2 = jax.nn.relu(_dot2(intra, Waff_ref[...]) + baff_ref[...]) + intra
    Kc_all = _dot2(intra2, Wck_ref[...])

    E_l = [_dg(Qc_all[u * NN:(u + 1) * NN],
               Kc_all[u * R:(u + 1) * R], dnT) * nmask for u in US]
    e_l = [_dot_splitL(E_l[u], Pc, dnL) * _INV_SCALE for u in US]
    w_l = []
    for u in US:
        e = jnp.where(maskf_ref[u] > 0, e_l[u], -1e9)
        e = e - jnp.max(e, axis=-1, keepdims=True)
        we = jnp.exp(e)
        w_l.append(we / jnp.sum(we, axis=-1, keepdims=True))
    ws_l = [_split(w_l[u]) for u in US]
    wR_l = [(_dg(ws_l[u][0], Pc, dnT) + _dg(ws_l[u][1], Pc, dnT)) * nmask
            for u in US]
    for u in US:
        out_ref[u] = _dg(wR_l[u], intra2[u * R:(u + 1) * R], dnL)


@jax.jit
def _sue_pallas(h0, cand, graph, maskf, onehot, Pn, Pc, nmask,
                W_gcn, b_gcn, W_K, W_Q, b_Q, W_aff, b_aff, W_ck, W_cq, b_cq):
    grid = (B // BB,)
    data_spec3 = lambda s1, s2: pl.BlockSpec((BB, s1, s2), lambda i: (i, 0, 0))
    w_spec = lambda shape: pl.BlockSpec(shape, lambda i: (0,) * len(shape))
    return pl.pallas_call(
        _sue_kernel,
        grid=grid,
        in_specs=[
            data_spec3(NP, D),
            data_spec3(NN, D),
            data_spec3(NP, NP),
            data_spec3(1, C),
            data_spec3(C, H),
            w_spec((R, NN)),
            w_spec((R, C)),
            w_spec((NN, R)),
            w_spec((L, D, D)),
            w_spec((L, 1, D)),
            w_spec((D, AD)),
            w_spec((D, AD)),
            w_spec((1, AD)),
            w_spec((D, D)),
            w_spec((1, D)),
            w_spec((D, AD)),
            w_spec((D, AD)),
            w_spec((1, AD)),
        ],
        out_specs=data_spec3(NN, D),
        out_shape=jax.ShapeDtypeStruct((B, NN, D), jnp.float32),
    )(h0, cand, graph, maskf, onehot, Pn, Pc, nmask,
      W_gcn, b_gcn, W_K, W_Q, b_Q, W_aff, b_aff, W_ck, W_cq, b_cq)


def kernel(history_embedding, candidate_news_representation, user_history_graph,
           user_history_category_mask, user_history_category_indices,
           proxy_node_embedding, W_gcn, b_gcn, W_K, W_Q, b_Q, W_aff, b_aff,
           W_ck, W_cq, b_cq):
    h0 = jnp.concatenate(
        [history_embedding,
         jnp.broadcast_to(proxy_node_embedding[None], (B, CATN, D)),
         jnp.zeros((B, NP - NODES, D), jnp.float32)], axis=1)
    graph = jnp.pad(user_history_graph,
                    ((0, 0), (0, NP - NODES), (0, NP - NODES)))
    maskf = user_history_category_mask.at[:, -1].set(1)
    maskf = (maskf > 0).astype(jnp.float32).reshape(B, 1, C)
    idx = user_history_category_indices.astype(jnp.int32)
    onehot = (idx[:, None, :] == jnp.arange(C, dtype=jnp.int32)[None, :, None]
              ).astype(jnp.float32)
    r = jnp.arange(R, dtype=jnp.int32)
    valid = (r < NN * C).astype(jnp.float32)
    Pn = ((r[:, None] // C == jnp.arange(NN, dtype=jnp.int32)[None, :])
          .astype(jnp.float32) * valid[:, None])
    Pc = ((r[:, None] % C == jnp.arange(C, dtype=jnp.int32)[None, :])
          .astype(jnp.float32) * valid[:, None])
    nmask = Pn.T
    return _sue_pallas(
        h0, candidate_news_representation, graph,
        maskf, onehot, Pn, Pc, nmask,
        W_gcn, b_gcn.reshape(L, 1, D), W_K, W_Q,
        b_Q.reshape(1, AD), W_aff, b_aff.reshape(1, D), W_ck, W_cq,
        b_cq.reshape(1, AD))


# speedup vs baseline: 3.2443x; 1.1765x over previous
import jax
import jax.numpy as jnp
from jax.experimental import pallas as pl
from jax.experimental.pallas import tpu as pltpu

B = 256
NN = 5
H = 50
CATN = 18
C = CATN + 1
D = 400
AD = 128
NODES = H + CATN
NP = 72
L = 2
R = 96
BB = 8

_INV_SCALE = 1.0 / (AD ** 0.5)


def _dg(x, w, dn):
    return jax.lax.dot_general(x, w, dn, preferred_element_type=jnp.float32)


def _split(x):
    hi = x.astype(jnp.bfloat16).astype(jnp.float32)
    return hi, x - hi


def _dot_splitL(x, oh, dn):
    x_hi, x_lo = _split(x)
    return _dg(x_hi, oh, dn) + _dg(x_lo, oh, dn)


def _dot2(x, w):
    return jax.lax.dot_general(x, w, (((1,), (0,)), ((), ())),
                               preferred_element_type=jnp.float32)


def _sue_kernel(hist_ref, cand_ref, graph_ref, mask_ref, idx_ref,
                proxy_ref, Pn_ref, Pc_ref, nmask_ref,
                Wg_ref, bg_ref, WK_ref, WQ_ref, bQ_ref,
                Waff_ref, baff_ref, Wck_ref, Wcq_ref, bcq_ref, out_ref,
                nodes_scr, agg_scr):
    proxy = proxy_ref[...]
    for u in range(BB):
        nodes_scr[u * NP:u * NP + H] = hist_ref[u]
        nodes_scr[u * NP + H:u * NP + NODES] = proxy
    h0 = nodes_scr[...]

    g = h0
    for l in range(L):
        for u in range(BB):
            agg_scr[u * NP:u * NP + NODES] = _dg(
                graph_ref[u], g[u * NP:u * NP + NODES],
                (((1,), (0,)), ((), ())))
        g = g + jax.nn.relu(_dot2(agg_scr[...], Wg_ref[l]) + bg_ref[l])
    gfa = g + h0
    gfa_hi, gfa_lo = _split(gfa)

    K_all = _dot2(gfa, WK_ref[...])
    cand = cand_ref[...].reshape(BB * NN, D)
    Q_all = _dot2(cand, WQ_ref[...]) + bQ_ref[...]
    Qc_all = _dot2(cand, Wcq_ref[...]) + bcq_ref[...]

    Pn = Pn_ref[...]
    Pc = Pc_ref[...]
    nmask = nmask_ref[...]
    dnL = (((1,), (0,)), ((), ()))
    dnT = (((1,), (1,)), ((), ()))
    US = range(BB)

    cat_iota = jax.lax.broadcasted_iota(jnp.int32, (C, H), 0)

    a_l = [_dg(Q_all[u * NN:(u + 1) * NN],
               K_all[u * NP:u * NP + H], dnT) * _INV_SCALE for u in US]
    oh_l = [(cat_iota == idx_ref[u]).astype(jnp.float32) for u in US]
    M_l = [jnp.max(jnp.where(oh_l[u][None, :, :] > 0,
                             a_l[u][:, None, :], -1e30), axis=-1) for u in US]
    m_l = [_dg(M_l[u], oh_l[u], dnL) for u in US]
    ex_l = [jnp.exp(a_l[u] - m_l[u]) for u in US]
    exs_l = [_split(ex_l[u]) for u in US]
    ssum_l = [_dg(exs_l[u][0], oh_l[u], dnT)
              + _dg(exs_l[u][1], oh_l[u], dnT) for u in US]
    sss_l = [_split(ssum_l[u]) for u in US]
    den_l = [_dg(sss_l[u][0], oh_l[u], dnL)
             + _dg(sss_l[u][1], oh_l[u], dnL) + 1e-12 for u in US]
    al_l = [_split(ex_l[u] / den_l[u]) for u in US]
    cf_l = [_dg(Pc, oh_l[u], dnL) for u in US]
    wfh_l = [_dg(Pn, al_l[u][0], dnL) * cf_l[u] for u in US]
    wfl_l = [_dg(Pn, al_l[u][1], dnL) * cf_l[u] for u in US]
    intra_list = [
        _dg(wfh_l[u], gfa_hi[u * NP:u * NP + H], dnL)
        + _dg(wfh_l[u], gfa_lo[u * NP:u * NP + H], dnL)
        + _dg(wfl_l[u], gfa_hi[u * NP:u * NP + H], dnL) for u in US]

    intra = jnp.concatenate(intra_list, axis=0)
    intra2 = jax.nn.relu(_dot2(intra, Waff_ref[...]) + baff_ref[...]) + intra
    Kc_all = _dot2(intra2, Wck_ref[...])

    E_l = [_dg(Qc_all[u * NN:(u + 1) * NN],
               Kc_all[u * R:(u + 1) * R], dnT) * nmask for u in US]
    e_l = [_dot_splitL(E_l[u], Pc, dnL) * _INV_SCALE for u in US]
    cw_iota = jax.lax.broadcasted_iota(jnp.int32, (1, C), 1)
    w_l = []
    for u in US:
        keep = (mask_ref[u] > 0) | (cw_iota == C - 1)
        e = jnp.where(keep, e_l[u], -1e9)
        e = e - jnp.max(e, axis=-1, keepdims=True)
        we = jnp.exp(e)
        w_l.append(we / jnp.sum(we, axis=-1, keepdims=True))
    ws_l = [_split(w_l[u]) for u in US]
    wR_l = [(_dg(ws_l[u][0], Pc, dnT) + _dg(ws_l[u][1], Pc, dnT)) * nmask
            for u in US]
    for u in US:
        out_ref[u] = _dg(wR_l[u], intra2[u * R:(u + 1) * R], dnL)


@jax.jit
def _sue_pallas(hist, cand, graph, mask, idx, proxy, Pn, Pc, nmask,
                W_gcn, b_gcn, W_K, W_Q, b_Q, W_aff, b_aff, W_ck, W_cq, b_cq):
    grid = (B // BB,)
    data_spec3 = lambda s1, s2: pl.BlockSpec((BB, s1, s2), lambda i: (i, 0, 0))
    w_spec = lambda shape: pl.BlockSpec(shape, lambda i: (0,) * len(shape))
    return pl.pallas_call(
        _sue_kernel,
        grid=grid,
        in_specs=[
            data_spec3(H, D),
            data_spec3(NN, D),
            data_spec3(NODES, NODES),
            data_spec3(1, C),
            data_spec3(1, H),
            w_spec((CATN, D)),
            w_spec((R, NN)),
            w_spec((R, C)),
            w_spec((NN, R)),
            w_spec((L, D, D)),
            w_spec((L, 1, D)),
            w_spec((D, AD)),
            w_spec((D, AD)),
            w_spec((1, AD)),
            w_spec((D, D)),
            w_spec((1, D)),
            w_spec((D, AD)),
            w_spec((D, AD)),
            w_spec((1, AD)),
        ],
        out_specs=data_spec3(NN, D),
        out_shape=jax.ShapeDtypeStruct((B, NN, D), jnp.float32),
        scratch_shapes=[pltpu.VMEM((BB * NP, D), jnp.float32),
                        pltpu.VMEM((BB * NP, D), jnp.float32)],
    )(hist, cand, graph, mask, idx, proxy, Pn, Pc, nmask,
      W_gcn, b_gcn, W_K, W_Q, b_Q, W_aff, b_aff, W_ck, W_cq, b_cq)


def kernel(history_embedding, candidate_news_representation, user_history_graph,
           user_history_category_mask, user_history_category_indices,
           proxy_node_embedding, W_gcn, b_gcn, W_K, W_Q, b_Q, W_aff, b_aff,
           W_ck, W_cq, b_cq):
    mask = user_history_category_mask.astype(jnp.int32).reshape(B, 1, C)
    idx = user_history_category_indices.astype(jnp.int32).reshape(B, 1, H)
    r = jnp.arange(R, dtype=jnp.int32)
    valid = (r < NN * C).astype(jnp.float32)
    Pn = ((r[:, None] // C == jnp.arange(NN, dtype=jnp.int32)[None, :])
          .astype(jnp.float32) * valid[:, None])
    Pc = ((r[:, None] % C == jnp.arange(C, dtype=jnp.int32)[None, :])
          .astype(jnp.float32) * valid[:, None])
    nmask = Pn.T
    return _sue_pallas(
        history_embedding, candidate_news_representation, user_history_graph,
        mask, idx, proxy_node_embedding, Pn, Pc, nmask,
        W_gcn, b_gcn.reshape(L, 1, D), W_K, W_Q,
        b_Q.reshape(1, AD), W_aff, b_aff.reshape(1, D), W_ck, W_cq,
        b_cq.reshape(1, AD))


# speedup vs baseline: 3.5139x; 1.0831x over previous
import jax
import jax.numpy as jnp
from jax.experimental import pallas as pl
from jax.experimental.pallas import tpu as pltpu

B = 256
NN = 5
H = 50
CATN = 18
C = CATN + 1
D = 400
AD = 128
NODES = H + CATN
NP = 72
L = 2
R = 96
BB = 16

_INV_SCALE = 1.0 / (AD ** 0.5)


def _dg(x, w, dn):
    return jax.lax.dot_general(x, w, dn, preferred_element_type=jnp.float32)


def _split(x):
    hi = x.astype(jnp.bfloat16).astype(jnp.float32)
    return hi, x - hi


def _dot_splitL(x, oh, dn):
    x_hi, x_lo = _split(x)
    return _dg(x_hi, oh, dn) + _dg(x_lo, oh, dn)


def _dot2(x, w):
    return jax.lax.dot_general(x, w, (((1,), (0,)), ((), ())),
                               preferred_element_type=jnp.float32)


def _sue_kernel(hist_ref, cand_ref, graph_ref, mask_ref, idx_ref,
                proxy_ref, Pn_ref, Pc_ref, nmask_ref,
                Wg_ref, bg_ref, WK_ref, WQ_ref, bQ_ref,
                Waff_ref, baff_ref, Wck_ref, Wcq_ref, bcq_ref, out_ref,
                nodes_scr, agg_scr):
    proxy = proxy_ref[...]
    for u in range(BB):
        nodes_scr[u * NP:u * NP + H] = hist_ref[u]
        nodes_scr[u * NP + H:u * NP + NODES] = proxy
    h0 = nodes_scr[...]

    g = h0
    for l in range(L):
        for u in range(BB):
            agg_scr[u * NP:u * NP + NODES] = _dg(
                graph_ref[u], g[u * NP:u * NP + NODES],
                (((1,), (0,)), ((), ())))
        g = g + jax.nn.relu(_dot2(agg_scr[...], Wg_ref[l]) + bg_ref[l])
    gfa = g + h0
    gfa_hi, gfa_lo = _split(gfa)

    K_all = _dot2(gfa, WK_ref[...])
    cand = cand_ref[...].reshape(BB * NN, D)
    Q_all = _dot2(cand, WQ_ref[...]) + bQ_ref[...]
    Qc_all = _dot2(cand, Wcq_ref[...]) + bcq_ref[...]

    Pn = Pn_ref[...]
    Pc = Pc_ref[...]
    nmask = nmask_ref[...]
    dnL = (((1,), (0,)), ((), ()))
    dnT = (((1,), (1,)), ((), ()))
    US = range(BB)

    cat_iota = jax.lax.broadcasted_iota(jnp.int32, (C, H), 0)

    a_l = [_dg(Q_all[u * NN:(u + 1) * NN],
               K_all[u * NP:u * NP + H], dnT) * _INV_SCALE for u in US]
    oh_l = [(cat_iota == idx_ref[u]).astype(jnp.float32) for u in US]
    M_l = [jnp.max(jnp.where(oh_l[u][None, :, :] > 0,
                             a_l[u][:, None, :], -1e30), axis=-1) for u in US]
    m_l = [_dg(M_l[u], oh_l[u], dnL) for u in US]
    ex_l = [jnp.exp(a_l[u] - m_l[u]) for u in US]
    exs_l = [_split(ex_l[u]) for u in US]
    ssum_l = [_dg(exs_l[u][0], oh_l[u], dnT)
              + _dg(exs_l[u][1], oh_l[u], dnT) for u in US]
    sss_l = [_split(ssum_l[u]) for u in US]
    den_l = [_dg(sss_l[u][0], oh_l[u], dnL)
             + _dg(sss_l[u][1], oh_l[u], dnL) + 1e-12 for u in US]
    al_l = [_split(ex_l[u] / den_l[u]) for u in US]
    cf_l = [_dg(Pc, oh_l[u], dnL) for u in US]
    wfh_l = [_dg(Pn, al_l[u][0], dnL) * cf_l[u] for u in US]
    wfl_l = [_dg(Pn, al_l[u][1], dnL) * cf_l[u] for u in US]
    intra_list = [
        _dg(wfh_l[u], gfa_hi[u * NP:u * NP + H], dnL)
        + _dg(wfh_l[u], gfa_lo[u * NP:u * NP + H], dnL)
        + _dg(wfl_l[u], gfa_hi[u * NP:u * NP + H], dnL) for u in US]

    intra = jnp.concatenate(intra_list, axis=0)
    intra2 = jax.nn.relu(_dot2(intra, Waff_ref[...]) + baff_ref[...]) + intra
    Kc_all = _dot2(intra2, Wck_ref[...])

    E_l = [_dg(Qc_all[u * NN:(u + 1) * NN],
               Kc_all[u * R:(u + 1) * R], dnT) * nmask for u in US]
    e_l = [_dot_splitL(E_l[u], Pc, dnL) * _INV_SCALE for u in US]
    cw_iota = jax.lax.broadcasted_iota(jnp.int32, (1, C), 1)
    w_l = []
    for u in US:
        keep = (mask_ref[u] > 0) | (cw_iota == C - 1)
        e = jnp.where(keep, e_l[u], -1e9)
        e = e - jnp.max(e, axis=-1, keepdims=True)
        we = jnp.exp(e)
        w_l.append(we / jnp.sum(we, axis=-1, keepdims=True))
    ws_l = [_split(w_l[u]) for u in US]
    wR_l = [(_dg(ws_l[u][0], Pc, dnT) + _dg(ws_l[u][1], Pc, dnT)) * nmask
            for u in US]
    for u in US:
        out_ref[u] = _dg(wR_l[u], intra2[u * R:(u + 1) * R], dnL)


@jax.jit
def _sue_pallas(hist, cand, graph, mask, idx, proxy, Pn, Pc, nmask,
                W_gcn, b_gcn, W_K, W_Q, b_Q, W_aff, b_aff, W_ck, W_cq, b_cq):
    grid = (B // BB,)
    data_spec3 = lambda s1, s2: pl.BlockSpec((BB, s1, s2), lambda i: (i, 0, 0))
    w_spec = lambda shape: pl.BlockSpec(shape, lambda i: (0,) * len(shape))
    return pl.pallas_call(
        _sue_kernel,
        grid=grid,
        in_specs=[
            data_spec3(H, D),
            data_spec3(NN, D),
            data_spec3(NODES, NODES),
            data_spec3(1, C),
            data_spec3(1, H),
            w_spec((CATN, D)),
            w_spec((R, NN)),
            w_spec((R, C)),
            w_spec((NN, R)),
            w_spec((L, D, D)),
            w_spec((L, 1, D)),
            w_spec((D, AD)),
            w_spec((D, AD)),
            w_spec((1, AD)),
            w_spec((D, D)),
            w_spec((1, D)),
            w_spec((D, AD)),
            w_spec((D, AD)),
            w_spec((1, AD)),
        ],
        out_specs=data_spec3(NN, D),
        out_shape=jax.ShapeDtypeStruct((B, NN, D), jnp.float32),
        scratch_shapes=[pltpu.VMEM((BB * NP, D), jnp.float32),
                        pltpu.VMEM((BB * NP, D), jnp.float32)],
    )(hist, cand, graph, mask, idx, proxy, Pn, Pc, nmask,
      W_gcn, b_gcn, W_K, W_Q, b_Q, W_aff, b_aff, W_ck, W_cq, b_cq)


def kernel(history_embedding, candidate_news_representation, user_history_graph,
           user_history_category_mask, user_history_category_indices,
           proxy_node_embedding, W_gcn, b_gcn, W_K, W_Q, b_Q, W_aff, b_aff,
           W_ck, W_cq, b_cq):
    mask = user_history_category_mask.astype(jnp.int32).reshape(B, 1, C)
    idx = user_history_category_indices.astype(jnp.int32).reshape(B, 1, H)
    r = jnp.arange(R, dtype=jnp.int32)
    valid = (r < NN * C).astype(jnp.float32)
    Pn = ((r[:, None] // C == jnp.arange(NN, dtype=jnp.int32)[None, :])
          .astype(jnp.float32) * valid[:, None])
    Pc = ((r[:, None] % C == jnp.arange(C, dtype=jnp.int32)[None, :])
          .astype(jnp.float32) * valid[:, None])
    nmask = Pn.T
    return _sue_pallas(
        history_embedding, candidate_news_representation, user_history_graph,
        mask, idx, proxy_node_embedding, Pn, Pc, nmask,
        W_gcn, b_gcn.reshape(L, 1, D), W_K, W_Q,
        b_Q.reshape(1, AD), W_aff, b_aff.reshape(1, D), W_ck, W_cq,
        b_cq.reshape(1, AD))


# speedup vs baseline: 3.6759x; 1.0461x over previous
import jax
import jax.numpy as jnp
from jax.experimental import pallas as pl
from jax.experimental.pallas import tpu as pltpu

B = 256
NN = 5
H = 50
CATN = 18
C = CATN + 1
D = 400
AD = 128
NODES = H + CATN
NP = 72
L = 2
R = 96
BB = 32

_INV_SCALE = 1.0 / (AD ** 0.5)


def _dg(x, w, dn):
    return jax.lax.dot_general(x, w, dn, preferred_element_type=jnp.float32)


def _split(x):
    hi = x.astype(jnp.bfloat16).astype(jnp.float32)
    return hi, x - hi


def _dot_splitL(x, oh, dn):
    x_hi, x_lo = _split(x)
    return _dg(x_hi, oh, dn) + _dg(x_lo, oh, dn)


def _dot2(x, w):
    return jax.lax.dot_general(x, w, (((1,), (0,)), ((), ())),
                               preferred_element_type=jnp.float32)


def _sue_kernel(hist_ref, cand_ref, graph_ref, mask_ref, idx_ref,
                proxy_ref, Pn_ref, Pc_ref, nmask_ref,
                Wg_ref, bg_ref, WK_ref, WQ_ref, bQ_ref,
                Waff_ref, baff_ref, Wck_ref, Wcq_ref, bcq_ref, out_ref,
                nodes_scr, agg_scr):
    proxy = proxy_ref[...]
    for u in range(BB):
        nodes_scr[u * NP:u * NP + H] = hist_ref[u]
        nodes_scr[u * NP + H:u * NP + NODES] = proxy
    h0 = nodes_scr[...]

    g = h0
    for l in range(L):
        for u in range(BB):
            agg_scr[u * NP:u * NP + NODES] = _dg(
                graph_ref[u], g[u * NP:u * NP + NODES],
                (((1,), (0,)), ((), ())))
        g = g + jax.nn.relu(_dot2(agg_scr[...], Wg_ref[l]) + bg_ref[l])
    gfa = g + h0
    gfa_hi, gfa_lo = _split(gfa)

    K_all = _dot2(gfa, WK_ref[...])
    cand = cand_ref[...].reshape(BB * NN, D)
    Q_all = _dot2(cand, WQ_ref[...]) + bQ_ref[...]
    Qc_all = _dot2(cand, Wcq_ref[...]) + bcq_ref[...]

    Pn = Pn_ref[...]
    Pc = Pc_ref[...]
    nmask = nmask_ref[...]
    dnL = (((1,), (0,)), ((), ()))
    dnT = (((1,), (1,)), ((), ()))
    US = range(BB)

    cat_iota = jax.lax.broadcasted_iota(jnp.int32, (C, H), 0)

    a_l = [_dg(Q_all[u * NN:(u + 1) * NN],
               K_all[u * NP:u * NP + H], dnT) * _INV_SCALE for u in US]
    oh_l = [(cat_iota == idx_ref[u]).astype(jnp.float32) for u in US]
    M_l = [jnp.max(jnp.where(oh_l[u][None, :, :] > 0,
                             a_l[u][:, None, :], -1e30), axis=-1) for u in US]
    m_l = [_dg(M_l[u], oh_l[u], dnL) for u in US]
    ex_l = [jnp.exp(a_l[u] - m_l[u]) for u in US]
    exs_l = [_split(ex_l[u]) for u in US]
    ssum_l = [_dg(exs_l[u][0], oh_l[u], dnT)
              + _dg(exs_l[u][1], oh_l[u], dnT) for u in US]
    sss_l = [_split(ssum_l[u]) for u in US]
    den_l = [_dg(sss_l[u][0], oh_l[u], dnL)
             + _dg(sss_l[u][1], oh_l[u], dnL) + 1e-12 for u in US]
    al_l = [_split(ex_l[u] / den_l[u]) for u in US]
    cf_l = [_dg(Pc, oh_l[u], dnL) for u in US]
    wfh_l = [_dg(Pn, al_l[u][0], dnL) * cf_l[u] for u in US]
    wfl_l = [_dg(Pn, al_l[u][1], dnL) * cf_l[u] for u in US]
    intra_list = [
        _dg(wfh_l[u], gfa_hi[u * NP:u * NP + H], dnL)
        + _dg(wfh_l[u], gfa_lo[u * NP:u * NP + H], dnL)
        + _dg(wfl_l[u], gfa_hi[u * NP:u * NP + H], dnL) for u in US]

    intra = jnp.concatenate(intra_list, axis=0)
    intra2 = jax.nn.relu(_dot2(intra, Waff_ref[...]) + baff_ref[...]) + intra
    Kc_all = _dot2(intra2, Wck_ref[...])

    E_l = [_dg(Qc_all[u * NN:(u + 1) * NN],
               Kc_all[u * R:(u + 1) * R], dnT) * nmask for u in US]
    e_l = [_dot_splitL(E_l[u], Pc, dnL) * _INV_SCALE for u in US]
    cw_iota = jax.lax.broadcasted_iota(jnp.int32, (1, C), 1)
    w_l = []
    for u in US:
        keep = (mask_ref[u] > 0) | (cw_iota == C - 1)
        e = jnp.where(keep, e_l[u], -1e9)
        e = e - jnp.max(e, axis=-1, keepdims=True)
        we = jnp.exp(e)
        w_l.append(we / jnp.sum(we, axis=-1, keepdims=True))
    ws_l = [_split(w_l[u]) for u in US]
    wR_l = [(_dg(ws_l[u][0], Pc, dnT) + _dg(ws_l[u][1], Pc, dnT)) * nmask
            for u in US]
    for u in US:
        out_ref[u] = _dg(wR_l[u], intra2[u * R:(u + 1) * R], dnL)


@jax.jit
def _sue_pallas(hist, cand, graph, mask, idx, proxy, Pn, Pc, nmask,
                W_gcn, b_gcn, W_K, W_Q, b_Q, W_aff, b_aff, W_ck, W_cq, b_cq):
    grid = (B // BB,)
    data_spec3 = lambda s1, s2: pl.BlockSpec((BB, s1, s2), lambda i: (i, 0, 0))
    w_spec = lambda shape: pl.BlockSpec(shape, lambda i: (0,) * len(shape))
    return pl.pallas_call(
        _sue_kernel,
        grid=grid,
        in_specs=[
            data_spec3(H, D),
            data_spec3(NN, D),
            data_spec3(NODES, NODES),
            data_spec3(1, C),
            data_spec3(1, H),
            w_spec((CATN, D)),
            w_spec((R, NN)),
            w_spec((R, C)),
            w_spec((NN, R)),
            w_spec((L, D, D)),
            w_spec((L, 1, D)),
            w_spec((D, AD)),
            w_spec((D, AD)),
            w_spec((1, AD)),
            w_spec((D, D)),
            w_spec((1, D)),
            w_spec((D, AD)),
            w_spec((D, AD)),
            w_spec((1, AD)),
        ],
        out_specs=data_spec3(NN, D),
        out_shape=jax.ShapeDtypeStruct((B, NN, D), jnp.float32),
        scratch_shapes=[pltpu.VMEM((BB * NP, D), jnp.float32),
                        pltpu.VMEM((BB * NP, D), jnp.float32)],
    )(hist, cand, graph, mask, idx, proxy, Pn, Pc, nmask,
      W_gcn, b_gcn, W_K, W_Q, b_Q, W_aff, b_aff, W_ck, W_cq, b_cq)


def kernel(history_embedding, candidate_news_representation, user_history_graph,
           user_history_category_mask, user_history_category_indices,
           proxy_node_embedding, W_gcn, b_gcn, W_K, W_Q, b_Q, W_aff, b_aff,
           W_ck, W_cq, b_cq):
    mask = user_history_category_mask.astype(jnp.int32).reshape(B, 1, C)
    idx = user_history_category_indices.astype(jnp.int32).reshape(B, 1, H)
    r = jnp.arange(R, dtype=jnp.int32)
    valid = (r < NN * C).astype(jnp.float32)
    Pn = ((r[:, None] // C == jnp.arange(NN, dtype=jnp.int32)[None, :])
          .astype(jnp.float32) * valid[:, None])
    Pc = ((r[:, None] % C == jnp.arange(C, dtype=jnp.int32)[None, :])
          .astype(jnp.float32) * valid[:, None])
    nmask = Pn.T
    return _sue_pallas(
        history_embedding, candidate_news_representation, user_history_graph,
        mask, idx, proxy_node_embedding, Pn, Pc, nmask,
        W_gcn, b_gcn.reshape(L, 1, D), W_K, W_Q,
        b_Q.reshape(1, AD), W_aff, b_aff.reshape(1, D), W_ck, W_cq,
        b_cq.reshape(1, AD))
